# baseline (speedup 1.0000x reference)
import jax
import jax.numpy as jnp
from jax import lax
from jax.experimental import pallas as pl
from jax.experimental.pallas import tpu as pltpu
from jax.experimental.pallas import tpu_sc as plsc

N = 10000
E = 320000
D = 128
H = 2
C = 64
ED = 32
SCALE = 0.125

NC = 2
NS = 16
EPT = E // NS
B = 32
NB = EPT // B
RPT = 624
RZ = 80

BN = 2000
f32 = jnp.float32


def _proj_body(x_ref, wq_ref, bq_ref, wk_ref, bk_ref, wv_ref, bv_ref, we_ref,
               ws_ref, bs_ref, qg_ref, kv_ref, s_ref):
  h = pl.program_id(0)
  xb = x_ref[...]
  q = xb @ wq_ref[...] + bq_ref[...]
  k = xb @ wk_ref[...] + bk_ref[...]
  v = xb @ wv_ref[...] + bv_ref[...]
  s_ref[...] = xb @ ws_ref[...] + bs_ref[...]
  we = we_ref[...]
  g0 = lax.dot_general(q[:, 0:C], we[:, 0:C], (((1,), (1,)), ((), ())))
  g1 = lax.dot_general(q[:, C:2 * C], we[:, C:2 * C], (((1,), (1,)), ((), ())))
  qh = jnp.where(h == 0, q[:, 0:C], q[:, C:2 * C])
  kh = jnp.where(h == 0, k[:, 0:C], k[:, C:2 * C])
  vh = jnp.where(h == 0, v[:, 0:C], v[:, C:2 * C])
  gh = jnp.where(h == 0, g0, g1)
  qg_ref[...] = jnp.concatenate([qh, gh, jnp.zeros((BN, ED), f32)], axis=1)
  kv_ref[...] = jnp.concatenate([kh, vh], axis=1)


def _project(x, Wq, bq, Wk, bk, Wv, bv, We, Wskip, bskip):
  full = lambda shape: pl.BlockSpec(shape, lambda h, i: (0, 0))
  rowx = pl.BlockSpec((BN, D), lambda h, i: (i, 0))
  rowh = pl.BlockSpec((BN, D), lambda h, i: (h * (N // BN) + i, 0))
  return pl.pallas_call(
      _proj_body,
      grid=(H, N // BN),
      in_specs=[
          rowx, full((D, D)), full((1, D)), full((D, D)), full((1, D)),
          full((D, D)), full((1, D)), full((ED, D)), full((D, D)),
          full((1, D)),
      ],
      out_specs=[rowh, rowh, rowx],
      out_shape=[
          jax.ShapeDtypeStruct((H * N, D), f32),
          jax.ShapeDtypeStruct((H * N, D), f32),
          jax.ShapeDtypeStruct((N, D), f32),
      ],
  )(x, Wq, bq.reshape(1, D), Wk, bk.reshape(1, D), Wv, bv.reshape(1, D), We,
    Wskip, bskip.reshape(1, D))


def _edge_body(qgt, kvt, tt, mt, srct, dstt, out_hbm,
               qgr0, kvr0, etr0, emr0, stg0,
               qgr1, kvr1, etr1, emr1, stg1,
               qgr2, kvr2, etr2, emr2, stg2,
               rs0, rd0, sg0, dg0, ds0,
               rs1, rd1, sg1, dg1, ds1,
               rs2, rd2, sg2, dg2, ds2,
               abuf, accsp,
               sq0, sk0, st0, sm0, ss0, sxs0, sxd0,
               sq1, sk1, st1, sm1, ss1, sxs1, sxd1,
               sq2, sk2, st2, sm2, ss2, sxs2, sxd2):
  c = lax.axis_index("c")
  s = lax.axis_index("s")
  cn = c * N
  ii = lax.iota(jnp.int32, 16)
  zero = ii.astype(f32) * 0.0

  @pl.loop(0, B)
  def _fill_z(j):
    for ch in range(D // 16):
      stg0[j, pl.ds(ch * 16, 16)] = zero
      stg1[j, pl.ds(ch * 16, 16)] = zero
      stg2[j, pl.ds(ch * 16, 16)] = zero

  row0 = s * RPT

  @pl.loop(0, RPT // (2 * B) + 1)
  def _zero_spmem(r):
    pltpu.sync_copy(stg0, accsp.at[pl.ds(row0 + r * B, B)])
    pltpu.sync_copy(stg1, accsp.at[pl.ds(row0 + (RPT // (2 * B) + 1 + r) * B,
                                         B)])

  plsc.subcore_barrier()

  base = s * EPT

  def issue_idx(bb, rs, rd, sxs, sxd):
    off = base + bb * B
    pltpu.async_copy(srct.at[pl.ds(off, B)], rs, sxs)
    pltpu.async_copy(dstt.at[pl.ds(off, B)], rd, sxd)

  def wait_idx(bb, rs, rd, sxs, sxd):
    off = base + bb * B
    pltpu.make_async_copy(srct.at[pl.ds(off, B)], rs, sxs).wait()
    pltpu.make_async_copy(dstt.at[pl.ds(off, B)], rd, sxd).wait()

  def fill_gidx(rs, rd, sg, dg):
    for i in range(B // 16):
      sg[pl.ds(i * 16, 16)] = rs[pl.ds(i * 16, 16)] + cn
      dg[pl.ds(i * 16, 16)] = rd[pl.ds(i * 16, 16)] + cn

  def fill_sidx(dg, dsb):
    for i in range(B // 16):
      dsb[pl.ds(i * 16, 16)] = dg[pl.ds(i * 16, 16)] - cn

  def issue_gathers(bb, sg, dg, qgr, kvr, etr, emr, sq, sk, st, sm):
    off = base + bb * B
    pltpu.async_copy(qgt.at[dg], qgr, sq)
    pltpu.async_copy(kvt.at[sg], kvr, sk)
    pltpu.async_copy(tt.at[pl.ds(off, B)], etr, st)
    pltpu.async_copy(mt.at[pl.ds(off, B)], emr, sm)

  def wait_gathers(sg, dg, qgr, kvr, etr, emr, sq, sk, st, sm, off):
    pltpu.make_async_copy(qgt.at[dg], qgr, sq).wait()
    pltpu.make_async_copy(kvt.at[sg], kvr, sk).wait()
    pltpu.make_async_copy(tt.at[pl.ds(off, B)], etr, st).wait()
    pltpu.make_async_copy(mt.at[pl.ds(off, B)], emr, sm).wait()

  iix16 = ii * 16

  def compute_block(qgr, kvr, etr, emr, stg):
    @pl.loop(0, B // 16)
    def _group(g):
      j0 = g * 16
      for jj in range(16):
        j = j0 + jj
        acc = qgr[j, pl.ds(0, 16)] * kvr[j, pl.ds(0, 16)]
        for ch in range(1, 4):
          acc += qgr[j, pl.ds(ch * 16, 16)] * kvr[j, pl.ds(ch * 16, 16)]
        acc += etr[j] * qgr[j, pl.ds(C, 16)]
        acc += emr[j] * qgr[j, pl.ds(C + 16, 16)]
        plsc.store_scatter(abuf, [iix16 + jj], acc)
      t0 = abuf[pl.ds(0, 16)] + abuf[pl.ds(16, 16)]
      t1 = abuf[pl.ds(32, 16)] + abuf[pl.ds(48, 16)]
      t2 = abuf[pl.ds(64, 16)] + abuf[pl.ds(80, 16)]
      t3 = abuf[pl.ds(96, 16)] + abuf[pl.ds(112, 16)]
      t4 = abuf[pl.ds(128, 16)] + abuf[pl.ds(144, 16)]
      t5 = abuf[pl.ds(160, 16)] + abuf[pl.ds(176, 16)]
      t6 = abuf[pl.ds(192, 16)] + abuf[pl.ds(208, 16)]
      t7 = abuf[pl.ds(224, 16)] + abuf[pl.ds(240, 16)]
      al = ((t0 + t1) + (t2 + t3)) + ((t4 + t5) + (t6 + t7))
      p = jnp.exp(al * SCALE)
      for jj in range(16):
        j = j0 + jj
        pb = jnp.full((16,), p[jj], f32)
        for ch in range(4):
          stg[j, pl.ds(ch * 16, 16)] = pb * kvr[j, pl.ds(C + ch * 16, 16)]
        stg[j, pl.ds(64, 16)] = pb * etr[j]
        stg[j, pl.ds(80, 16)] = pb * emr[j]
        stg[j, pl.ds(96, 16)] = jnp.where(ii == 0, pb, 0.0)

  S0 = (qgr0, kvr0, etr0, emr0, stg0, rs0, rd0, sg0, dg0, ds0,
        sq0, sk0, st0, sm0, ss0, sxs0, sxd0)
  S1 = (qgr1, kvr1, etr1, emr1, stg1, rs1, rd1, sg1, dg1, ds1,
        sq1, sk1, st1, sm1, ss1, sxs1, sxd1)
  S2 = (qgr2, kvr2, etr2, emr2, stg2, rs2, rd2, sg2, dg2, ds2,
        sq2, sk2, st2, sm2, ss2, sxs2, sxd2)

  def step(x, S, g_more, g_idxw, g_idxi):
    (qgr, kvr, etr, emr, stg, rs, rd, sg, dg, ds,
     sq, sk, st, sm, ss, sxs, sxd) = S
    wait_gathers(sg, dg, qgr, kvr, etr, emr, sq, sk, st, sm, base + x * B)

    @pl.when(x >= 3)
    def _ws():
      pltpu.make_async_copy(stg, accsp.at[ds], ss).wait()

    fill_sidx(dg, ds)

    @pl.when(g_idxw)
    def _wx():
      wait_idx(x + 3, rs, rd, sxs, sxd)
      fill_gidx(rs, rd, sg, dg)

    @pl.when(g_idxi)
    def _ix():
      issue_idx(x + 6, rs, rd, sxs, sxd)

    compute_block(qgr, kvr, etr, emr, stg)
    pltpu.async_copy(stg, accsp.at[ds], ss, add=True)

    @pl.when(g_more)
    def _ig():
      issue_gathers(x + 3, sg, dg, qgr, kvr, etr, emr, sq, sk, st, sm)

  for r, S in ((0, S0), (1, S1), (2, S2)):
    (qgr, kvr, etr, emr, stg, rs, rd, sg, dg, ds,
     sq, sk, st, sm, ss, sxs, sxd) = S
    pltpu.sync_copy(srct.at[pl.ds(base + r * B, B)], rs)
    pltpu.sync_copy(dstt.at[pl.ds(base + r * B, B)], rd)
    fill_gidx(rs, rd, sg, dg)
    issue_gathers(r, sg, dg, qgr, kvr, etr, emr, sq, sk, st, sm)
  issue_idx(3, rs0, rd0, sxs0, sxd0)
  issue_idx(4, rs1, rd1, sxs1, sxd1)
  issue_idx(5, rs2, rd2, sxs2, sxd2)

  NR = NB // 3

  @pl.loop(0, NR)
  def _t(t):
    x0 = t * 3
    step(x0, S0, t < NR, t < NR, t < NR - 1)
    step(x0 + 1, S1, t < NR - 1, t < NR - 1, t < NR - 2)
    step(x0 + 2, S2, t < NR - 1, t < NR - 1, t < NR - 2)

  wait_gathers(sg0, dg0, qgr0, kvr0, etr0, emr0, sq0, sk0, st0, sm0,
               base + (NB - 1) * B)
  pltpu.make_async_copy(stg0, accsp.at[ds0], ss0).wait()
  fill_sidx(dg0, ds0)
  compute_block(qgr0, kvr0, etr0, emr0, stg0)
  pltpu.async_copy(stg0, accsp.at[ds0], ss0, add=True)
  pltpu.make_async_copy(stg1, accsp.at[ds1], ss1).wait()
  pltpu.make_async_copy(stg2, accsp.at[ds2], ss2).wait()
  pltpu.make_async_copy(stg0, accsp.at[ds0], ss0).wait()

  plsc.subcore_barrier()

  @pl.when(s < NS - 1)
  def _flush_body():
    pltpu.sync_copy(accsp.at[pl.ds(row0, RPT)],
                    out_hbm.at[c, pl.ds(row0, RPT)])

  @pl.when(s == NS - 1)
  def _flush_tail():
    pltpu.sync_copy(accsp.at[pl.ds((NS - 1) * RPT, N - (NS - 1) * RPT)],
                    out_hbm.at[c, pl.ds((NS - 1) * RPT, N - (NS - 1) * RPT)])


def _edge_pass(qg_tab, kv_tab, t, msg, src, dst):
  kfn = pl.kernel(
      _edge_body,
      out_type=jax.ShapeDtypeStruct((NC, N, D), f32),
      mesh=plsc.VectorSubcoreMesh(core_axis_name="c", subcore_axis_name="s"),
      compiler_params=pltpu.CompilerParams(needs_layout_passes=False,
                                           use_tc_tiling_on_sc=False),
      scratch_types=(
          [
              pltpu.VMEM((B, D), f32),
              pltpu.VMEM((B, D), f32),
              pltpu.VMEM((B, ED // 2), f32),
              pltpu.VMEM((B, ED // 2), f32),
              pltpu.VMEM((B, D), f32),
          ] * 3
          + [pltpu.VMEM((B,), jnp.int32)] * 15
          + [
              pltpu.VMEM((256,), f32),
              pltpu.VMEM_SHARED((N, D), f32),
          ] + [pltpu.SemaphoreType.DMA] * 21),
  )
  return kfn(qg_tab, kv_tab, t, msg, src, dst)


def _comb_body(ovp_ref, skip_ref, we_ref, out_ref):
  ov0 = ovp_ref[0]
  ov1 = ovp_ref[1]
  we = we_ref[...]
  e0 = lax.dot_general(ov0[:, C:C + ED], we[:, 0:C], (((1,), (0,)), ((), ())))
  e1 = lax.dot_general(ov1[:, C:C + ED], we[:, C:2 * C],
                       (((1,), (0,)), ((), ())))
  o0 = (ov0[:, 0:C] + e0) / (ov0[:, 96:97] + 1e-16)
  o1 = (ov1[:, 0:C] + e1) / (ov1[:, 96:97] + 1e-16)
  out_ref[...] = jnp.concatenate([o0, o1], axis=1) + skip_ref[...]


def _combine(ovp, skip, We):
  return pl.pallas_call(
      _comb_body,
      grid=(N // BN,),
      in_specs=[
          pl.BlockSpec((NC, BN, D), lambda i: (0, i, 0)),
          pl.BlockSpec((BN, D), lambda i: (i, 0)),
          pl.BlockSpec((ED, D), lambda i: (0, 0)),
      ],
      out_specs=pl.BlockSpec((BN, D), lambda i: (i, 0)),
      out_shape=jax.ShapeDtypeStruct((N, D), f32),
  )(ovp, skip, We)


def kernel(x, last_update, edge_index, t, msg, Wq, bq, Wk, bk, Wv, bv, We,
           Wskip, bskip):
  del last_update
  src = edge_index[0]
  dst = edge_index[1]
  qg_tab, kv_tab, skip = _project(x, Wq, bq, Wk, bk, Wv, bv, We, Wskip, bskip)
  ovp = _edge_pass(qg_tab, kv_tab, t, msg, src, dst)
  return _combine(ovp, skip, We)


# speedup vs baseline: 1.3304x; 1.3304x over previous
import jax
import jax.numpy as jnp
from jax import lax
from jax.experimental import pallas as pl
from jax.experimental.pallas import tpu as pltpu
from jax.experimental.pallas import tpu_sc as plsc

N = 10000
E = 320000
D = 128
H = 2
C = 64
ED = 32
SCALE = 0.125

NC = 2
NS = 16
EPT = E // NS
B = 32
NB = EPT // B
RPT = 624
RZ = 80

BN = 2000
f32 = jnp.float32


def _proj_body(x_ref, wq_ref, bq_ref, wk_ref, bk_ref, wv_ref, bv_ref, we_ref,
               ws_ref, bs_ref, qg_ref, kv_ref, s_ref):
  h = pl.program_id(0)
  xb = x_ref[...]
  q = xb @ wq_ref[...] + bq_ref[...]
  k = xb @ wk_ref[...] + bk_ref[...]
  v = xb @ wv_ref[...] + bv_ref[...]
  s_ref[...] = xb @ ws_ref[...] + bs_ref[...]
  we = we_ref[...]
  g0 = lax.dot_general(q[:, 0:C], we[:, 0:C], (((1,), (1,)), ((), ())))
  g1 = lax.dot_general(q[:, C:2 * C], we[:, C:2 * C], (((1,), (1,)), ((), ())))
  qh = jnp.where(h == 0, q[:, 0:C], q[:, C:2 * C])
  kh = jnp.where(h == 0, k[:, 0:C], k[:, C:2 * C])
  vh = jnp.where(h == 0, v[:, 0:C], v[:, C:2 * C])
  gh = jnp.where(h == 0, g0, g1)
  qg_ref[...] = jnp.concatenate([qh, gh, jnp.zeros((BN, ED), f32)], axis=1)
  kv_ref[...] = jnp.concatenate([kh, vh], axis=1)


def _project(x, Wq, bq, Wk, bk, Wv, bv, We, Wskip, bskip):
  full = lambda shape: pl.BlockSpec(shape, lambda h, i: (0, 0))
  rowx = pl.BlockSpec((BN, D), lambda h, i: (i, 0))
  rowh = pl.BlockSpec((BN, D), lambda h, i: (h * (N // BN) + i, 0))
  return pl.pallas_call(
      _proj_body,
      grid=(H, N // BN),
      in_specs=[
          rowx, full((D, D)), full((1, D)), full((D, D)), full((1, D)),
          full((D, D)), full((1, D)), full((ED, D)), full((D, D)),
          full((1, D)),
      ],
      out_specs=[rowh, rowh, rowx],
      out_shape=[
          jax.ShapeDtypeStruct((H * N, D), f32),
          jax.ShapeDtypeStruct((H * N, D), f32),
          jax.ShapeDtypeStruct((N, D), f32),
      ],
  )(x, Wq, bq.reshape(1, D), Wk, bk.reshape(1, D), Wv, bv.reshape(1, D), We,
    Wskip, bskip.reshape(1, D))


def _edge_body(qgt, kvt, tt, mt, srct, dstt, out_hbm,
               qgr0, kvr0, etr0, emr0, stg0,
               qgr1, kvr1, etr1, emr1, stg1,
               rs0, rd0, sg0, dg0, ds0,
               rs1, rd1, sg1, dg1, ds1,
               abuf, accsp,
               sq0, sk0, st0, sm0, ss0, sxs0, sxd0,
               sq1, sk1, st1, sm1, ss1, sxs1, sxd1):
  c = lax.axis_index("c")
  s = lax.axis_index("s")
  cn = c * N
  ii = lax.iota(jnp.int32, 16)
  zero = ii.astype(f32) * 0.0

  @pl.loop(0, B)
  def _fill_z(j):
    for ch in range(D // 16):
      stg0[j, pl.ds(ch * 16, 16)] = zero
      stg1[j, pl.ds(ch * 16, 16)] = zero

  row0 = s * RPT

  @pl.loop(0, RPT // (2 * B) + 1)
  def _zero_spmem(r):
    pltpu.sync_copy(stg0, accsp.at[pl.ds(row0 + r * B, B)])
    pltpu.sync_copy(stg1, accsp.at[pl.ds(row0 + (RPT // (2 * B) + 1 + r) * B,
                                         B)])

  plsc.subcore_barrier()

  base = s * EPT

  def issue_idx(bb, rs, rd, sxs, sxd):
    off = base + bb * B
    pltpu.async_copy(srct.at[pl.ds(off, B)], rs, sxs)
    pltpu.async_copy(dstt.at[pl.ds(off, B)], rd, sxd)

  def wait_idx(bb, rs, rd, sxs, sxd):
    off = base + bb * B
    pltpu.make_async_copy(srct.at[pl.ds(off, B)], rs, sxs).wait()
    pltpu.make_async_copy(dstt.at[pl.ds(off, B)], rd, sxd).wait()

  def fill_gidx(rs, rd, sg, dg):
    for i in range(B // 16):
      sg[pl.ds(i * 16, 16)] = rs[pl.ds(i * 16, 16)] + cn
      dg[pl.ds(i * 16, 16)] = rd[pl.ds(i * 16, 16)] + cn

  def fill_sidx(dg, dsb):
    for i in range(B // 16):
      dsb[pl.ds(i * 16, 16)] = dg[pl.ds(i * 16, 16)] - cn

  def issue_gathers(bb, sg, dg, qgr, kvr, etr, emr, sq, sk, st, sm):
    off = base + bb * B
    pltpu.async_copy(qgt.at[dg], qgr, sq)
    pltpu.async_copy(kvt.at[sg], kvr, sk)
    pltpu.async_copy(tt.at[pl.ds(off, B)], etr, st)
    pltpu.async_copy(mt.at[pl.ds(off, B)], emr, sm)

  def wait_gathers(sg, dg, qgr, kvr, etr, emr, sq, sk, st, sm, off):
    pltpu.make_async_copy(qgt.at[dg], qgr, sq).wait()
    pltpu.make_async_copy(kvt.at[sg], kvr, sk).wait()
    pltpu.make_async_copy(tt.at[pl.ds(off, B)], etr, st).wait()
    pltpu.make_async_copy(mt.at[pl.ds(off, B)], emr, sm).wait()

  iix16 = ii * 16

  def compute_block(qgr, kvr, etr, emr, stg):
    @pl.loop(0, B // 16)
    def _group(g):
      j0 = g * 16
      for jj in range(16):
        j = j0 + jj
        acc = qgr[j, pl.ds(0, 16)] * kvr[j, pl.ds(0, 16)]
        for ch in range(1, 4):
          acc += qgr[j, pl.ds(ch * 16, 16)] * kvr[j, pl.ds(ch * 16, 16)]
        acc += etr[j] * qgr[j, pl.ds(C, 16)]
        acc += emr[j] * qgr[j, pl.ds(C + 16, 16)]
        plsc.store_scatter(abuf, [iix16 + jj], acc)
      t0 = abuf[pl.ds(0, 16)] + abuf[pl.ds(16, 16)]
      t1 = abuf[pl.ds(32, 16)] + abuf[pl.ds(48, 16)]
      t2 = abuf[pl.ds(64, 16)] + abuf[pl.ds(80, 16)]
      t3 = abuf[pl.ds(96, 16)] + abuf[pl.ds(112, 16)]
      t4 = abuf[pl.ds(128, 16)] + abuf[pl.ds(144, 16)]
      t5 = abuf[pl.ds(160, 16)] + abuf[pl.ds(176, 16)]
      t6 = abuf[pl.ds(192, 16)] + abuf[pl.ds(208, 16)]
      t7 = abuf[pl.ds(224, 16)] + abuf[pl.ds(240, 16)]
      al = ((t0 + t1) + (t2 + t3)) + ((t4 + t5) + (t6 + t7))
      p = jnp.exp(al * SCALE)
      for jj in range(16):
        j = j0 + jj
        pb = jnp.full((16,), p[jj], f32)
        for ch in range(4):
          stg[j, pl.ds(ch * 16, 16)] = pb * kvr[j, pl.ds(C + ch * 16, 16)]
        stg[j, pl.ds(64, 16)] = pb * etr[j]
        stg[j, pl.ds(80, 16)] = pb * emr[j]
        stg[j, pl.ds(96, 16)] = jnp.where(ii == 0, pb, 0.0)

  S0 = (qgr0, kvr0, etr0, emr0, stg0, rs0, rd0, sg0, dg0, ds0,
        sq0, sk0, st0, sm0, ss0, sxs0, sxd0)
  S1 = (qgr1, kvr1, etr1, emr1, stg1, rs1, rd1, sg1, dg1, ds1,
        sq1, sk1, st1, sm1, ss1, sxs1, sxd1)

  def step(x, S, g_gather, g_idxw, g_idxi):
    (qgr, kvr, etr, emr, stg, rs, rd, sg, dg, ds,
     sq, sk, st, sm, ss, sxs, sxd) = S
    wait_gathers(sg, dg, qgr, kvr, etr, emr, sq, sk, st, sm, base + x * B)

    @pl.when(x >= 2)
    def _ws():
      pltpu.make_async_copy(stg, accsp.at[ds], ss).wait()

    fill_sidx(dg, ds)

    @pl.when(g_idxw)
    def _wx():
      wait_idx(x + 2, rs, rd, sxs, sxd)
      fill_gidx(rs, rd, sg, dg)

    @pl.when(g_idxi)
    def _ix():
      issue_idx(x + 4, rs, rd, sxs, sxd)

    compute_block(qgr, kvr, etr, emr, stg)
    pltpu.async_copy(stg, accsp.at[ds], ss, add=True)

    @pl.when(g_gather)
    def _ig():
      issue_gathers(x + 2, sg, dg, qgr, kvr, etr, emr, sq, sk, st, sm)

  for r, S in ((0, S0), (1, S1)):
    (qgr, kvr, etr, emr, stg, rs, rd, sg, dg, ds,
     sq, sk, st, sm, ss, sxs, sxd) = S
    pltpu.sync_copy(srct.at[pl.ds(base + r * B, B)], rs)
    pltpu.sync_copy(dstt.at[pl.ds(base + r * B, B)], rd)
    fill_gidx(rs, rd, sg, dg)
    issue_gathers(r, sg, dg, qgr, kvr, etr, emr, sq, sk, st, sm)
  issue_idx(2, rs0, rd0, sxs0, sxd0)
  issue_idx(3, rs1, rd1, sxs1, sxd1)

  NR = NB // 2

  @pl.loop(0, NR)
  def _t(t):
    x0 = t * 2
    step(x0, S0, t < NR, t < NR, t < NR - 1)
    step(x0 + 1, S1, t < NR - 1, t < NR - 1, t < NR - 2)

  wait_gathers(sg0, dg0, qgr0, kvr0, etr0, emr0, sq0, sk0, st0, sm0,
               base + (NB - 1) * B)
  pltpu.make_async_copy(stg0, accsp.at[ds0], ss0).wait()
  fill_sidx(dg0, ds0)
  compute_block(qgr0, kvr0, etr0, emr0, stg0)
  pltpu.async_copy(stg0, accsp.at[ds0], ss0, add=True)
  pltpu.make_async_copy(stg1, accsp.at[ds1], ss1).wait()
  pltpu.make_async_copy(stg0, accsp.at[ds0], ss0).wait()

  plsc.subcore_barrier()

  @pl.when(s < NS - 1)
  def _flush_body():
    pltpu.sync_copy(accsp.at[pl.ds(row0, RPT)],
                    out_hbm.at[c, pl.ds(row0, RPT)])

  @pl.when(s == NS - 1)
  def _flush_tail():
    pltpu.sync_copy(accsp.at[pl.ds((NS - 1) * RPT, N - (NS - 1) * RPT)],
                    out_hbm.at[c, pl.ds((NS - 1) * RPT, N - (NS - 1) * RPT)])


def _edge_pass(qg_tab, kv_tab, t, msg, src, dst):
  kfn = pl.kernel(
      _edge_body,
      out_type=jax.ShapeDtypeStruct((NC, N, D), f32),
      mesh=plsc.VectorSubcoreMesh(core_axis_name="c", subcore_axis_name="s"),
      compiler_params=pltpu.CompilerParams(needs_layout_passes=False,
                                           use_tc_tiling_on_sc=False),
      scratch_types=(
          [
              pltpu.VMEM((B, D), f32),
              pltpu.VMEM((B, D), f32),
              pltpu.VMEM((B, ED // 2), f32),
              pltpu.VMEM((B, ED // 2), f32),
              pltpu.VMEM((B, D), f32),
          ] * 2
          + [pltpu.VMEM((B,), jnp.int32)] * 10
          + [
              pltpu.VMEM((256,), f32),
              pltpu.VMEM_SHARED((N, D), f32),
          ] + [pltpu.SemaphoreType.DMA] * 14),
  )
  return kfn(qg_tab, kv_tab, t, msg, src, dst)


def _comb_body(ovp_ref, skip_ref, we_ref, out_ref):
  ov0 = ovp_ref[0]
  ov1 = ovp_ref[1]
  we = we_ref[...]
  e0 = lax.dot_general(ov0[:, C:C + ED], we[:, 0:C], (((1,), (0,)), ((), ())))
  e1 = lax.dot_general(ov1[:, C:C + ED], we[:, C:2 * C],
                       (((1,), (0,)), ((), ())))
  o0 = (ov0[:, 0:C] + e0) / (ov0[:, 96:97] + 1e-16)
  o1 = (ov1[:, 0:C] + e1) / (ov1[:, 96:97] + 1e-16)
  out_ref[...] = jnp.concatenate([o0, o1], axis=1) + skip_ref[...]


def _combine(ovp, skip, We):
  return pl.pallas_call(
      _comb_body,
      grid=(N // BN,),
      in_specs=[
          pl.BlockSpec((NC, BN, D), lambda i: (0, i, 0)),
          pl.BlockSpec((BN, D), lambda i: (i, 0)),
          pl.BlockSpec((ED, D), lambda i: (0, 0)),
      ],
      out_specs=pl.BlockSpec((BN, D), lambda i: (i, 0)),
      out_shape=jax.ShapeDtypeStruct((N, D), f32),
  )(ovp, skip, We)


def kernel(x, last_update, edge_index, t, msg, Wq, bq, Wk, bk, Wv, bv, We,
           Wskip, bskip):
  del last_update
  src = edge_index[0]
  dst = edge_index[1]
  qg_tab, kv_tab, skip = _project(x, Wq, bq, Wk, bk, Wv, bv, We, Wskip, bskip)
  ovp = _edge_pass(qg_tab, kv_tab, t, msg, src, dst)
  return _combine(ovp, skip, We)
